# baseline (speedup 1.0000x reference)
import jax
import jax.numpy as jnp
from jax import lax
from jax.experimental import pallas as pl
from jax.experimental.pallas import tpu as pltpu
from jax.experimental.pallas import tpu_sc as plsc

N_FIELDS = 13
VOCAB = 100000
DIM = 16
BATCH = 16384

NC, NS = 2, 16
NW = NC * NS
BPW = BATCH // NW
NB = 128
NCH = BPW // NB
K = N_FIELDS * NB


def _body(xt_hbm, ut_hbm, it_hbm, out_hbm, xv, uidx, iidx, urows, irows,
          sem_g, sem_w):
    wid = lax.axis_index("s") * NC + lax.axis_index("c")

    for ch in range(NCH):
        base = wid * BPW + ch * NB

        pltpu.sync_copy(xt_hbm.at[:, pl.ds(base, NB)], xv)

        def build(s, _):
            off = s * 16
            f = off // NB
            vu = xv[f, pl.ds(off - f * NB, 16)]
            vi = xv[f + N_FIELDS, pl.ds(off - f * NB, 16)]
            uidx[pl.ds(off, 16)] = vu + f * VOCAB
            iidx[pl.ds(off, 16)] = vi + f * VOCAB
            return 0

        lax.fori_loop(0, K // 16, build, 0)

        gathers = []
        for f in range(N_FIELDS):
            gathers.append(pltpu.async_copy(
                ut_hbm.at[uidx.at[pl.ds(f * NB, NB)]],
                urows.at[pl.ds(f * NB, NB), :], sem_g))
            gathers.append(pltpu.async_copy(
                it_hbm.at[iidx.at[pl.ds(f * NB, NB)]],
                irows.at[pl.ds(f * NB, NB), :], sem_g))
        for g in gathers:
            g.wait()

        writes = []
        for f in range(N_FIELDS):
            writes.append(pltpu.async_copy(
                urows.at[pl.ds(f * NB, NB), :],
                out_hbm.at[pl.ds(base, NB), f, :], sem_w))
            writes.append(pltpu.async_copy(
                irows.at[pl.ds(f * NB, NB), :],
                out_hbm.at[pl.ds(base, NB), f + N_FIELDS, :], sem_w))
        for w in writes:
            w.wait()


@jax.jit
def kernel(x, user_tables, item_tables):
    xt = x.astype(jnp.int32).T
    uflat = user_tables.reshape(N_FIELDS * VOCAB, DIM)
    iflat = item_tables.reshape(N_FIELDS * VOCAB, DIM)

    mesh = plsc.VectorSubcoreMesh(
        core_axis_name="c", subcore_axis_name="s",
        num_cores=NC, num_subcores=NS)
    out3 = pl.kernel(
        _body,
        out_type=jax.ShapeDtypeStruct((BATCH, 2 * N_FIELDS, DIM), jnp.float32),
        mesh=mesh,
        compiler_params=pltpu.CompilerParams(use_tc_tiling_on_sc=False),
        scratch_types=[
            pltpu.VMEM((2 * N_FIELDS, NB), jnp.int32),
            pltpu.VMEM((K,), jnp.int32),
            pltpu.VMEM((K,), jnp.int32),
            pltpu.VMEM((K, DIM), jnp.float32),
            pltpu.VMEM((K, DIM), jnp.float32),
            pltpu.SemaphoreType.DMA,
            pltpu.SemaphoreType.DMA,
        ],
    )(xt, uflat, iflat)
    return out3.reshape(BATCH, 2 * N_FIELDS * DIM)


# speedup vs baseline: 1.1807x; 1.1807x over previous
import jax
import jax.numpy as jnp
from jax import lax
from jax.experimental import pallas as pl
from jax.experimental.pallas import tpu as pltpu
from jax.experimental.pallas import tpu_sc as plsc

N_FIELDS = 13
VOCAB = 100000
DIM = 16
BATCH = 16384

NC, NS = 2, 16
NW = NC * NS
BPW = BATCH // NW
NB = 128
NCH = BPW // NB
K = N_FIELDS * NB


def _body(xt_hbm, ut_hbm, it_hbm, out_hbm, xv, uidx, iidx,
          uo0, uo1, io0, io1, ur0, ur1, ir0, ir1, sg0, sg1, sw0, sw1):
    wid = lax.axis_index("s") * NC + lax.axis_index("c")
    uo, io = [uo0, uo1], [io0, io1]
    urows, irows = [ur0, ur1], [ir0, ir1]
    sg, sw = [sg0, sg1], [sw0, sw1]
    lanes26 = lax.iota(jnp.int32, 16) * 26

    prev_writes = [None, None]
    for ch in range(NCH):
        b = ch & 1
        base = wid * BPW + ch * NB

        pltpu.sync_copy(xt_hbm.at[:, pl.ds(base, NB)], xv)

        uo_b, io_b = uo[b], io[b]

        def build(s, _):
            off = s * 16
            f = off // NB
            r = off - f * NB
            vu = xv[f, pl.ds(r, 16)]
            vi = xv[f + N_FIELDS, pl.ds(r, 16)]
            uidx[pl.ds(off, 16)] = vu + f * VOCAB
            iidx[pl.ds(off, 16)] = vi + f * VOCAB
            ov = lanes26 + ((base + r) * 26 + f)
            uo_b[pl.ds(off, 16)] = ov
            io_b[pl.ds(off, 16)] = ov + N_FIELDS
            return 0

        lax.fori_loop(0, K // 16, build, 0)

        if prev_writes[b] is not None:
            for w in prev_writes[b]:
                w.wait()

        g_u = pltpu.async_copy(ut_hbm.at[uidx], urows[b], sg[b])
        g_i = pltpu.async_copy(it_hbm.at[iidx], irows[b], sg[b])
        g_u.wait()
        g_i.wait()

        w_u = pltpu.async_copy(urows[b], out_hbm.at[uo_b], sw[b])
        w_i = pltpu.async_copy(irows[b], out_hbm.at[io_b], sw[b])
        prev_writes[b] = (w_u, w_i)

    for pw in prev_writes:
        if pw is not None:
            for w in pw:
                w.wait()


@jax.jit
def kernel(x, user_tables, item_tables):
    xt = x.astype(jnp.int32).T
    uflat = user_tables.reshape(N_FIELDS * VOCAB, DIM)
    iflat = item_tables.reshape(N_FIELDS * VOCAB, DIM)

    mesh = plsc.VectorSubcoreMesh(
        core_axis_name="c", subcore_axis_name="s",
        num_cores=NC, num_subcores=NS)
    outf = pl.kernel(
        _body,
        out_type=jax.ShapeDtypeStruct((BATCH * 2 * N_FIELDS, DIM), jnp.float32),
        mesh=mesh,
        compiler_params=pltpu.CompilerParams(use_tc_tiling_on_sc=False),
        scratch_types=[
            pltpu.VMEM((2 * N_FIELDS, NB), jnp.int32),
            pltpu.VMEM((K,), jnp.int32),
            pltpu.VMEM((K,), jnp.int32),
            pltpu.VMEM((K,), jnp.int32),
            pltpu.VMEM((K,), jnp.int32),
            pltpu.VMEM((K,), jnp.int32),
            pltpu.VMEM((K,), jnp.int32),
            pltpu.VMEM((K, DIM), jnp.float32),
            pltpu.VMEM((K, DIM), jnp.float32),
            pltpu.VMEM((K, DIM), jnp.float32),
            pltpu.VMEM((K, DIM), jnp.float32),
            pltpu.SemaphoreType.DMA,
            pltpu.SemaphoreType.DMA,
            pltpu.SemaphoreType.DMA,
            pltpu.SemaphoreType.DMA,
        ],
    )(xt, uflat, iflat)
    return outf.reshape(BATCH, 2 * N_FIELDS * DIM)
